# Initial kernel scaffold; baseline (speedup 1.0000x reference)
import jax
import jax.numpy as jnp
from jax.experimental import pallas as pl


def kernel(c1w, c1b, c2w, c2b, f1w, f1b, f2w, f2b, f3w, f3b, x):
    raise NotImplementedError("write your pallas kernel here")


# speedup vs baseline: 12.6179x; 12.6179x over previous
import jax
import jax.numpy as jnp
from jax.experimental import pallas as pl
from jax.experimental.pallas import tpu as pltpu

_B = 32


def _net_kernel(x_ref, t1_ref, b1_ref, s1_ref, e1_ref, t2_ref, b2_ref,
                s2_ref, e2_ref, w1_ref, f1b_ref, w2_ref, f2b_ref, w3_ref,
                f3b_ref, o_ref):
    B = x_ref.shape[0]
    f32 = jnp.float32
    bf16 = jnp.bfloat16

    R1 = B * 48
    xb = x_ref[...].reshape(R1, 456).astype(bf16)
    acc = jnp.zeros((R1, 888), f32)
    z1 = jnp.zeros((4, 456), bf16)
    for dy in range(5):
        xs = xb if dy == 0 else jnp.concatenate([xb[dy:], z1[:dy]], axis=0)
        acc = acc + jnp.dot(xs, t1_ref[dy], preferred_element_type=f32)
    a = jnp.maximum(acc + b1_ref[...], 0.0).astype(bf16)
    am = jnp.maximum(a, jnp.concatenate([a[1:], jnp.zeros((1, 888), bf16)],
                                        axis=0))
    cm = jnp.maximum(am, jnp.concatenate([am[:, 1:],
                                          jnp.zeros((R1, 1), bf16)], axis=1))
    cp = jnp.dot(cm, s1_ref[...], preferred_element_type=f32).astype(bf16)
    x2 = jnp.dot(e1_ref[...], cp, preferred_element_type=f32).astype(bf16)

    R2 = B * 24
    acc2 = jnp.zeros((R2, 1120), f32)
    z2 = jnp.zeros((4, 444), bf16)
    for dy in range(5):
        xs = x2 if dy == 0 else jnp.concatenate([x2[dy:], z2[:dy]], axis=0)
        acc2 = acc2 + jnp.dot(xs, t2_ref[dy], preferred_element_type=f32)
    a2 = jnp.maximum(acc2 + b2_ref[...], 0.0).astype(bf16)
    am2 = jnp.maximum(a2, jnp.concatenate([a2[1:], jnp.zeros((1, 1120), bf16)],
                                          axis=0))
    cm2 = jnp.maximum(am2, jnp.concatenate([am2[:, 1:],
                                            jnp.zeros((R2, 1), bf16)], axis=1))
    p2 = jnp.dot(cm2, s2_ref[...], preferred_element_type=f32).astype(bf16)

    h = jnp.zeros((B, 120), f32)
    for k in range(9):
        fk = jnp.dot(e2_ref[k], p2, preferred_element_type=f32).astype(bf16)
        h = h + jnp.dot(fk, w1_ref[k], preferred_element_type=f32)
    h = jnp.maximum(h + f1b_ref[...], 0.0).astype(bf16)
    h = jnp.dot(h, w2_ref[...], preferred_element_type=f32) + f2b_ref[...]
    h = jnp.maximum(h, 0.0).astype(bf16)
    o_ref[...] = jnp.dot(h, w3_ref[...], preferred_element_type=f32) + f3b_ref[...]


def _toeplitz(wconv, wsrc, wout):
    kw = wconv.shape[3]
    w = jnp.arange(wsrc)
    wo = jnp.arange(wout)
    dx = jnp.arange(kw)
    mask = (w[None, :, None] == wo[None, None, :] + dx[:, None, None])
    t = jnp.einsum('xwv,ocdx->dcwov', mask.astype(jnp.float32), wconv)
    kh, cin, cout = wconv.shape[2], wconv.shape[1], wconv.shape[0]
    return t.reshape(kh, cin * wsrc, cout * wout).astype(jnp.bfloat16)


def _even_col_selector(nch, wout, wp):
    r = jnp.arange(nch * wout)
    c = jnp.arange(nch * wp)
    sel = (r[:, None] // wout == c[None, :] // wp) & (
        r[:, None] % wout == 2 * (c[None, :] % wp))
    return sel.astype(jnp.bfloat16)


def kernel(c1w, c1b, c2w, c2b, f1w, f1b, f2w, f2b, f3w, f3b, x):
    N = x.shape[0]
    B = _B
    f32 = jnp.float32
    bf16 = jnp.bfloat16

    xt = x.astype(f32).transpose(0, 2, 1, 3).reshape(N, 48, 3 * 152)
    t1 = _toeplitz(c1w.reshape(6, 3, 5, 5), 152, 148)
    t2 = _toeplitz(c2w.reshape(16, 6, 5, 5), 74, 70)
    s1 = _even_col_selector(6, 148, 74)
    s2 = _even_col_selector(16, 70, 35)
    r = jnp.arange(B * 24)
    c = jnp.arange(B * 48)
    e1 = (c[None, :] == (r[:, None] // 24) * 48
          + 2 * (r[:, None] % 24)).astype(bf16)
    kk = jnp.arange(9)
    bb = jnp.arange(B)
    cc = jnp.arange(B * 24)
    e2 = (cc[None, None, :] == bb[None, :, None] * 24
          + 2 * kk[:, None, None]).astype(bf16)
    b1r = jnp.repeat(c1b, 148)[None, :].astype(f32)
    b2r = jnp.repeat(c2b, 70)[None, :].astype(f32)
    w1r = f1w.reshape(16, 9, 35, 120).transpose(1, 0, 2, 3)
    w1r = w1r.reshape(9, 560, 120).astype(bf16)
    w2b = f2w.astype(bf16)
    w3b = f3w.astype(bf16)

    full = lambda arr: pl.BlockSpec(arr.shape, lambda n: (0,) * arr.ndim)
    out = pl.pallas_call(
        _net_kernel,
        out_shape=jax.ShapeDtypeStruct((N, 8), f32),
        grid=(N // B,),
        in_specs=[
            pl.BlockSpec((B, 48, 456), lambda n: (n, 0, 0)),
            full(t1), full(b1r), full(s1), full(e1),
            full(t2), full(b2r), full(s2), full(e2),
            full(w1r), full(f1b), full(w2b), full(f2b), full(w3b), full(f3b),
        ],
        out_specs=pl.BlockSpec((B, 8), lambda n: (n, 0)),
        compiler_params=pltpu.CompilerParams(
            dimension_semantics=("parallel",)),
    )(xt, t1, b1r, s1, e1, t2, b2r, s2, e2, w1r, f1b, w2b, f2b, w3b, f3b)
    return out


# speedup vs baseline: 14.1722x; 1.1232x over previous
import jax
import jax.numpy as jnp
from jax.experimental import pallas as pl
from jax.experimental.pallas import tpu as pltpu

_B = 64


def _rshift(v, s):
    if s == 0:
        return v
    return jnp.concatenate(
        [v[s:], jnp.zeros((s,) + v.shape[1:], v.dtype)], axis=0)


def _lshift1(v):
    return jnp.concatenate(
        [v[:, 1:], jnp.zeros((v.shape[0], 1), v.dtype)], axis=1)


def _net_kernel(x_ref, t1_ref, b1_ref, s1_ref, t2_ref, b2_ref, s2_ref,
                e2_ref, w1_ref, f1b_ref, w2_ref, f2b_ref, w3_ref, f3b_ref,
                o_ref):
    B = x_ref.shape[1] // 12
    R = B * 12
    f32 = jnp.float32
    bf16 = jnp.bfloat16

    xb = x_ref[...].astype(bf16)
    x0 = [xb[q] for q in range(4)]
    x1 = [_rshift(v, 1) for v in x0]
    acc = []
    for p in range(4):
        s = jnp.zeros((R, 888), f32)
        for dy in range(5):
            q, sh = (p + dy) % 4, (p + dy) // 4
            lhs = x0[q] if sh == 0 else x1[q]
            s = s + jnp.dot(lhs, t1_ref[dy], preferred_element_type=f32)
        acc.append(jnp.maximum(s + b1_ref[...], 0.0).astype(bf16))
    x2 = []
    for ve, vo in ((acc[0], acc[1]), (acc[2], acc[3])):
        cm = jnp.maximum(ve, vo)
        cm = jnp.maximum(cm, _lshift1(cm))
        x2.append(jnp.dot(cm, s1_ref[...],
                          preferred_element_type=f32).astype(bf16))

    x2s = [[x2[0], _rshift(x2[0], 1), _rshift(x2[0], 2)],
           [x2[1], _rshift(x2[1], 1), _rshift(x2[1], 2)]]
    a2 = []
    for e in range(2):
        s = jnp.zeros((R, 1120), f32)
        for dy in range(5):
            q, sh = (e + dy) % 2, (e + dy) // 2
            s = s + jnp.dot(x2s[q][sh], t2_ref[dy],
                            preferred_element_type=f32)
        a2.append(jnp.maximum(s + b2_ref[...], 0.0))
    fr = jnp.maximum(a2[0], a2[1]).astype(bf16)
    cm2 = jnp.maximum(fr, _lshift1(fr))
    p2 = jnp.dot(cm2, s2_ref[...], preferred_element_type=f32).astype(bf16)

    h = jnp.zeros((B, 120), f32)
    for k in range(9):
        fk = jnp.dot(e2_ref[k], p2, preferred_element_type=f32).astype(bf16)
        h = h + jnp.dot(fk, w1_ref[k], preferred_element_type=f32)
    h = jnp.maximum(h + f1b_ref[...], 0.0).astype(bf16)
    h = jnp.dot(h, w2_ref[...], preferred_element_type=f32) + f2b_ref[...]
    h = jnp.maximum(h, 0.0).astype(bf16)
    o_ref[...] = jnp.dot(h, w3_ref[...], preferred_element_type=f32) + f3b_ref[...]


def _toeplitz(wconv, wsrc, wout):
    kw = wconv.shape[3]
    w = jnp.arange(wsrc)
    wo = jnp.arange(wout)
    dx = jnp.arange(kw)
    mask = (w[None, :, None] == wo[None, None, :] + dx[:, None, None])
    t = jnp.einsum('xwv,ocdx->dcwov', mask.astype(jnp.float32), wconv)
    kh, cin, cout = wconv.shape[2], wconv.shape[1], wconv.shape[0]
    return t.reshape(kh, cin * wsrc, cout * wout).astype(jnp.bfloat16)


def _even_col_selector(nch, wout, wp):
    r = jnp.arange(nch * wout)
    c = jnp.arange(nch * wp)
    sel = (r[:, None] // wout == c[None, :] // wp) & (
        r[:, None] % wout == 2 * (c[None, :] % wp))
    return sel.astype(jnp.bfloat16)


def kernel(c1w, c1b, c2w, c2b, f1w, f1b, f2w, f2b, f3w, f3b, x):
    N = x.shape[0]
    B = _B
    f32 = jnp.float32
    bf16 = jnp.bfloat16

    xt = x.astype(f32).transpose(0, 2, 1, 3).reshape(N, 12, 4, 3 * 152)
    xq = xt.transpose(2, 0, 1, 3).reshape(4, N * 12, 3 * 152)

    t1 = _toeplitz(c1w.reshape(6, 3, 5, 5), 152, 148)
    t2 = _toeplitz(c2w.reshape(16, 6, 5, 5), 74, 70)
    s1 = _even_col_selector(6, 148, 74)
    s2 = _even_col_selector(16, 70, 35)
    kk = jnp.arange(9)
    bb = jnp.arange(B)
    cc = jnp.arange(B * 12)
    e2 = (cc[None, None, :] == bb[None, :, None] * 12
          + kk[:, None, None]).astype(bf16)
    b1r = jnp.repeat(c1b, 148)[None, :].astype(f32)
    b2r = jnp.repeat(c2b, 70)[None, :].astype(f32)
    w1r = f1w.reshape(16, 9, 35, 120).transpose(1, 0, 2, 3)
    w1r = w1r.reshape(9, 560, 120).astype(bf16)
    w2b = f2w.astype(bf16)
    w3b = f3w.astype(bf16)

    full = lambda arr: pl.BlockSpec(arr.shape, lambda n: (0,) * arr.ndim)
    out = pl.pallas_call(
        _net_kernel,
        out_shape=jax.ShapeDtypeStruct((N, 8), f32),
        grid=(N // B,),
        in_specs=[
            pl.BlockSpec((4, B * 12, 456), lambda n: (0, n, 0)),
            full(t1), full(b1r), full(s1),
            full(t2), full(b2r), full(s2),
            full(e2), full(w1r), full(f1b), full(w2b), full(f2b),
            full(w3b), full(f3b),
        ],
        out_specs=pl.BlockSpec((B, 8), lambda n: (n, 0)),
        compiler_params=pltpu.CompilerParams(
            dimension_semantics=("parallel",)),
    )(xq, t1, b1r, s1, t2, b2r, s2, e2, w1r, f1b, w2b, f2b, w3b, f3b)
    return out


# speedup vs baseline: 22.5557x; 1.5916x over previous
import jax
import jax.numpy as jnp
from jax.experimental import pallas as pl
from jax.experimental.pallas import tpu as pltpu

_B = 64

_C1_W = ((0, 85), (72, 80))
_C1_WO = (80, 76)
_C1_J = (40, 38)
_C2_WO = (36, 34)
_C2_J = (18, 17)


def _rshift(v, s):
    if s == 0:
        return v
    return jnp.concatenate(
        [v[s:], jnp.zeros((s,) + v.shape[1:], v.dtype)], axis=0)


def _lshift1(v):
    return jnp.concatenate(
        [v[:, 1:], jnp.zeros((v.shape[0], 1), v.dtype)], axis=1)


def _net_kernel(x_ref,
                t1a_ref, t1b_ref, b1a_ref, b1b_ref, s1a_ref, s1b_ref,
                t2a_ref, t2b_ref, b2a_ref, b2b_ref, s2a_ref, s2b_ref,
                e2_ref, w1_ref, f1b_ref, w2_ref, f2b_ref, w3_ref, f3b_ref,
                o_ref):
    B = x_ref.shape[0]
    R = B * 12
    f32 = jnp.float32
    bf16 = jnp.bfloat16
    t1 = (t1a_ref, t1b_ref)
    s1 = (s1a_ref, s1b_ref)
    b1 = (b1a_ref, b1b_ref)
    t2 = (t2a_ref, t2b_ref)
    s2 = (s2a_ref, s2b_ref)
    b2 = (b2a_ref, b2b_ref)

    xc = []
    vall = x_ref[...].astype(bf16)
    for p in range(4):
        vci = [vall[:, ci, :, p * 152:(p + 1) * 152].reshape(R, 152)
               for ci in range(3)]
        xc.append([
            jnp.concatenate([v[:, w0:w0 + wd] for v in vci], axis=1)
            for (w0, wd) in _C1_W])
    xs1 = [[c, _rshift(c, 1)] for c in
           (xc[0][0], xc[1][0], xc[2][0], xc[3][0])]
    xs2 = [[c, _rshift(c, 1)] for c in
           (xc[0][1], xc[1][1], xc[2][1], xc[3][1])]

    x2p = [[None, None], [None, None]]
    for c, xsc in ((0, xs1), (1, xs2)):
        ncol = 6 * _C1_WO[c]
        acc = []
        for p in range(4):
            s = jnp.zeros((R, ncol), f32)
            for dy in range(5):
                q, sh = (p + dy) % 4, (p + dy) // 4
                s = s + jnp.dot(xsc[q][sh], t1[c][dy],
                                preferred_element_type=f32)
            acc.append(jnp.maximum(s + b1[c][...], 0.0).astype(bf16))
        for e in range(2):
            cm = jnp.maximum(acc[2 * e], acc[2 * e + 1])
            cm = jnp.maximum(cm, _lshift1(cm))
            x2p[e][c] = jnp.dot(cm, s1[c][...],
                                preferred_element_type=f32).astype(bf16)

    x2s = [[[v, _rshift(v, 1), _rshift(v, 2)] for v in (x2p[0][c], x2p[1][c])]
           for c in range(2)]
    p2 = [None, None]
    for c in range(2):
        ncol = 16 * _C2_WO[c]
        a2 = []
        for e in range(2):
            s = jnp.zeros((R, ncol), f32)
            for dy in range(5):
                q, sh = (e + dy) % 2, (e + dy) // 2
                s = s + jnp.dot(x2s[c][q][sh], t2[c][dy],
                                preferred_element_type=f32)
            a2.append(jnp.maximum(s + b2[c][...], 0.0))
        fr = jnp.maximum(a2[0], a2[1]).astype(bf16)
        cm2 = jnp.maximum(fr, _lshift1(fr))
        p2[c] = jnp.dot(cm2, s2[c][...],
                        preferred_element_type=f32).astype(bf16)
    p2cat = jnp.concatenate(p2, axis=1)

    h = jnp.zeros((B, 120), f32)
    for k in range(9):
        fk = jnp.dot(e2_ref[k], p2cat, preferred_element_type=f32).astype(bf16)
        h = h + jnp.dot(fk, w1_ref[k], preferred_element_type=f32)
    h = jnp.maximum(h + f1b_ref[...], 0.0).astype(bf16)
    h = jnp.dot(h, w2_ref[...], preferred_element_type=f32) + f2b_ref[...]
    h = jnp.maximum(h, 0.0).astype(bf16)
    o_ref[...] = jnp.dot(h, w3_ref[...], preferred_element_type=f32) + f3b_ref[...]


def _toeplitz(wconv, wsrc, wout):
    kw = wconv.shape[3]
    w = jnp.arange(wsrc)
    wo = jnp.arange(wout)
    dx = jnp.arange(kw)
    mask = (w[None, :, None] == wo[None, None, :] + dx[:, None, None])
    t = jnp.einsum('xwv,ocdx->dcwov', mask.astype(jnp.float32), wconv)
    kh, cin, cout = wconv.shape[2], wconv.shape[1], wconv.shape[0]
    return t.reshape(kh, cin * wsrc, cout * wout).astype(jnp.bfloat16)


def _even_col_selector(nch, wout, wp):
    r = jnp.arange(nch * wout)
    c = jnp.arange(nch * wp)
    sel = (r[:, None] // wout == c[None, :] // wp) & (
        r[:, None] % wout == 2 * (c[None, :] % wp))
    return sel.astype(jnp.bfloat16)


def kernel(c1w, c1b, c2w, c2b, f1w, f1b, f2w, f2b, f3w, f3b, x):
    N = x.shape[0]
    B = _B
    f32 = jnp.float32
    bf16 = jnp.bfloat16

    xr = x.astype(f32).reshape(N, 3, 12, 4 * 152)

    w1c = c1w.reshape(6, 3, 5, 5)
    w2c = c2w.reshape(16, 6, 5, 5)
    t1a = _toeplitz(w1c, _C1_W[0][1], _C1_WO[0])
    t1b = _toeplitz(w1c, _C1_W[1][1], _C1_WO[1])
    t2a = _toeplitz(w2c, _C1_J[0], _C2_WO[0])
    t2b = _toeplitz(w2c, _C1_J[1], _C2_WO[1])
    s1a = _even_col_selector(6, _C1_WO[0], _C1_J[0])
    s1b = _even_col_selector(6, _C1_WO[1], _C1_J[1])
    s2a = _even_col_selector(16, _C2_WO[0], _C2_J[0])
    s2b = _even_col_selector(16, _C2_WO[1], _C2_J[1])
    b1a = jnp.repeat(c1b, _C1_WO[0])[None, :].astype(f32)
    b1b = jnp.repeat(c1b, _C1_WO[1])[None, :].astype(f32)
    b2a = jnp.repeat(c2b, _C2_WO[0])[None, :].astype(f32)
    b2b = jnp.repeat(c2b, _C2_WO[1])[None, :].astype(f32)
    kk = jnp.arange(9)
    bb = jnp.arange(B)
    cc = jnp.arange(B * 12)
    e2 = (cc[None, None, :] == bb[None, :, None] * 12
          + kk[:, None, None]).astype(bf16)
    wA = f1w.reshape(16, 9, 35, 120)
    part0 = wA[:, :, 0:18].transpose(1, 0, 2, 3).reshape(9, 16 * 18, 120)
    part1 = wA[:, :, 18:35].transpose(1, 0, 2, 3).reshape(9, 16 * 17, 120)
    w1r = jnp.concatenate([part0, part1], axis=1).astype(bf16)
    w2b = f2w.astype(bf16)
    w3b = f3w.astype(bf16)

    full = lambda arr: pl.BlockSpec(arr.shape, lambda n: (0,) * arr.ndim)
    out = pl.pallas_call(
        _net_kernel,
        out_shape=jax.ShapeDtypeStruct((N, 8), f32),
        grid=(N // B,),
        in_specs=[
            pl.BlockSpec((B, 3, 12, 608), lambda n: (n, 0, 0, 0)),
            full(t1a), full(t1b), full(b1a), full(b1b), full(s1a), full(s1b),
            full(t2a), full(t2b), full(b2a), full(b2b), full(s2a), full(s2b),
            full(e2), full(w1r), full(f1b), full(w2b), full(f2b),
            full(w3b), full(f3b),
        ],
        out_specs=pl.BlockSpec((B, 8), lambda n: (n, 0)),
        compiler_params=pltpu.CompilerParams(
            dimension_semantics=("parallel",)),
    )(xr, t1a, t1b, b1a, b1b, s1a, s1b,
      t2a, t2b, b2a, b2b, s2a, s2b, e2, w1r, f1b, w2b, f2b, w3b, f3b)
    return out
